# baseline (speedup 1.0000x reference)
import functools

import jax
import jax.numpy as jnp
from jax import lax
from jax.experimental import pallas as pl
from jax.experimental.pallas import tpu as pltpu
from jax.experimental.pallas import tpu_sc as plsc

BATCH = 16384
D = 64
NC = 2
NS = 16
NW = NC * NS
BPW = BATCH // NW
CHUNK = 128
NCH = BPW // CHUNK
IDX_ROWS = BATCH // CHUNK

_sc_mesh = plsc.VectorSubcoreMesh(core_axis_name="c", subcore_axis_name="s")


@functools.partial(
    pl.kernel,
    out_type=[jax.ShapeDtypeStruct((IDX_ROWS, CHUNK, D), jnp.float32)] * 4,
    mesh=_sc_mesh,
    compiler_params=pltpu.CompilerParams(use_tc_tiling_on_sc=False),
    scratch_types=[
        pltpu.VMEM((NCH, CHUNK), jnp.int32),
        pltpu.VMEM((NCH, CHUNK), jnp.int32),
        pltpu.VMEM((NCH, CHUNK, D), jnp.float32),
        pltpu.VMEM((NCH, CHUNK, D), jnp.float32),
        pltpu.VMEM((NCH, CHUNK, D), jnp.float32),
        pltpu.SemaphoreType.DMA,
        pltpu.SemaphoreType.DMA,
    ],
)
def _sc_gather(uidx_hbm, midx_hbm, eu_mf, em_mf, eu_mlp, em_mlp,
               ue_mf_out, me_mf_out, ue_mlp_out, me_mlp_out,
               uidx_v, midx_v, buf_a, buf_b, buf_c, gsem, ssem):
    wid = lax.axis_index("s") * NC + lax.axis_index("c")
    r0 = wid * NCH

    pltpu.sync_copy(uidx_hbm.at[pl.ds(r0, NCH)], uidx_v)
    pltpu.sync_copy(midx_hbm.at[pl.ds(r0, NCH)], midx_v)

    def fire(table, idx_v, buf):
        return [pltpu.async_copy(table.at[idx_v.at[j]], buf.at[j], gsem)
                for j in range(NCH)]

    def drain(descs):
        for d in descs:
            d.wait()

    g0 = fire(eu_mf, uidx_v, buf_a)
    g1 = fire(em_mf, midx_v, buf_b)
    drain(g0)
    s0 = pltpu.async_copy(buf_a, ue_mf_out.at[pl.ds(r0, NCH)], ssem)
    g2 = fire(eu_mlp, uidx_v, buf_c)
    drain(g1)
    s1 = pltpu.async_copy(buf_b, me_mf_out.at[pl.ds(r0, NCH)], ssem)
    s0.wait()
    g3 = fire(em_mlp, midx_v, buf_a)
    drain(g2)
    s2 = pltpu.async_copy(buf_c, ue_mlp_out.at[pl.ds(r0, NCH)], ssem)
    drain(g3)
    s3 = pltpu.async_copy(buf_a, me_mlp_out.at[pl.ds(r0, NCH)], ssem)
    s1.wait()
    s2.wait()
    s3.wait()


BB = 2048


def _tc_body(ue_mf, me_mf, ue_mlp, me_mlp, w1u, w1m, b1, w2, b2, w3, b3,
             w4, b4, wfa, wfb, bfv, out_ref):
    dot = functools.partial(jnp.dot, preferred_element_type=jnp.float32)
    h = jnp.maximum(dot(ue_mlp[...], w1u[...]) + dot(me_mlp[...], w1m[...])
                    + b1[...], 0.0)
    h = jnp.maximum(dot(h, w2[...]) + b2[...], 0.0)
    h = jnp.maximum(dot(h, w3[...]) + b3[...], 0.0)
    h = jnp.maximum(dot(h, w4[...]) + b4[...], 0.0)
    mf = ue_mf[...] * me_mf[...]
    p = (jnp.sum(mf * wfa[...], axis=1) + jnp.sum(h * wfb[...], axis=1)
         + bfv[0, 0])
    out_ref[...] = 1.0 / (1.0 + jnp.exp(-p))


def _full(shape):
    return pl.BlockSpec(shape, lambda i: tuple(0 for _ in shape))


_tc_mlp = pl.pallas_call(
    _tc_body,
    grid=(BATCH // BB,),
    in_specs=[
        pl.BlockSpec((BB, D), lambda i: (i, 0)),
        pl.BlockSpec((BB, D), lambda i: (i, 0)),
        pl.BlockSpec((BB, D), lambda i: (i, 0)),
        pl.BlockSpec((BB, D), lambda i: (i, 0)),
        _full((D, 128)), _full((D, 128)), _full((1, 128)),
        _full((128, D)), _full((1, D)),
        _full((D, 32)), _full((1, 32)),
        _full((32, 16)), _full((1, 16)),
        _full((1, D)), _full((1, 16)), _full((1, 1)),
    ],
    out_specs=pl.BlockSpec((BB,), lambda i: (i,)),
    out_shape=jax.ShapeDtypeStruct((BATCH,), jnp.float32),
)


@jax.jit
def kernel(user_indices, movie_indices, Eu_mf, Em_mf, Eu_mlp, Em_mlp,
           W1, b1, W2, b2, W3, b3, W4, b4, Wf, bf):
    uidx = user_indices.astype(jnp.int32).reshape(IDX_ROWS, CHUNK)
    midx = movie_indices.astype(jnp.int32).reshape(IDX_ROWS, CHUNK)
    ue_mf, me_mf, ue_mlp, me_mlp = _sc_gather(
        uidx, midx, Eu_mf, Em_mf, Eu_mlp, Em_mlp)
    ue_mf = ue_mf.reshape(BATCH, D)
    me_mf = me_mf.reshape(BATCH, D)
    ue_mlp = ue_mlp.reshape(BATCH, D)
    me_mlp = me_mlp.reshape(BATCH, D)
    return _tc_mlp(
        ue_mf, me_mf, ue_mlp, me_mlp,
        W1[:D], W1[D:], b1.reshape(1, 128),
        W2, b2.reshape(1, D),
        W3, b3.reshape(1, 32),
        W4, b4.reshape(1, 16),
        Wf[:D, 0].reshape(1, D), Wf[D:, 0].reshape(1, 16),
        (bf.reshape(1, 1)).astype(jnp.float32),
    )


# speedup vs baseline: 1.0051x; 1.0051x over previous
import functools

import jax
import jax.numpy as jnp
from jax import lax
from jax.experimental import pallas as pl
from jax.experimental.pallas import tpu as pltpu
from jax.experimental.pallas import tpu_sc as plsc

BATCH = 16384
D = 64
DR = 128
NC = 2
NS = 16
NW = NC * NS
BPW = BATCH // NW
CHUNK = 128
NCH = BPW // CHUNK
NCHUNKS = BATCH // CHUNK
NTAB = 4
NOPS = NTAB * NCH
RING = 6

_sc_mesh = plsc.VectorSubcoreMesh(core_axis_name="c", subcore_axis_name="s")


@functools.partial(
    pl.kernel,
    out_type=[jax.ShapeDtypeStruct((NCHUNKS, CHUNK, DR), jnp.float32)] * NTAB,
    mesh=_sc_mesh,
    scratch_types=[
        pltpu.VMEM((NCH, CHUNK), jnp.int32),
        pltpu.VMEM((NCH, CHUNK), jnp.int32),
        pltpu.VMEM((RING, CHUNK, DR), jnp.float32),
        pltpu.SemaphoreType.DMA,
        pltpu.SemaphoreType.DMA,
    ],
)
def _sc_gather(uidx_hbm, midx_hbm, eu_mf, em_mf, eu_mlp, em_mlp,
               ue_mf_out, me_mf_out, ue_mlp_out, me_mlp_out,
               uidx_v, midx_v, ring, gsem, ssem):
    wid = lax.axis_index("s") * NC + lax.axis_index("c")
    c0 = wid * NCH

    pltpu.sync_copy(uidx_hbm.at[wid], uidx_v)
    pltpu.sync_copy(midx_hbm.at[wid], midx_v)

    tables = (eu_mf, em_mf, eu_mlp, em_mlp)
    idxs = (uidx_v, midx_v, uidx_v, midx_v)
    outs = (ue_mf_out, me_mf_out, ue_mlp_out, me_mlp_out)

    gd = [None] * NOPS
    sd = [None] * NOPS
    for k in range(NOPS):
        t, j = k // NCH, k % NCH
        if k >= RING:
            sd[k - RING].wait()
        gd[k] = pltpu.async_copy(
            tables[t].at[idxs[t].at[j]], ring.at[k % RING], gsem)
        if k >= 1:
            kp = k - 1
            tp, jp = kp // NCH, kp % NCH
            gd[kp].wait()
            sd[kp] = pltpu.async_copy(
                ring.at[kp % RING], outs[tp].at[c0 + jp], ssem)
    gd[NOPS - 1].wait()
    sd[NOPS - 1] = pltpu.async_copy(
        ring.at[(NOPS - 1) % RING], outs[NTAB - 1].at[c0 + NCH - 1], ssem)
    for k in range(NOPS - RING, NOPS):
        sd[k].wait()


BB = 2048


def _tc_body(ue_mf, me_mf, ue_mlp, me_mlp, pu, pm, w1u, w1m, b1, w2, b2,
             w3, b3, w4, b4, wfa, wfb, bfv, out_ref):
    dot = functools.partial(jnp.dot, preferred_element_type=jnp.float32)

    up = pu[...] > 0.5
    mp = pm[...] > 0.5

    def sel(raw, p):
        return jnp.where(p, raw[:, D:], raw[:, :D])

    ue = sel(ue_mlp[...], up)
    me = sel(me_mlp[...], mp)
    h = jnp.maximum(dot(ue, w1u[...]) + dot(me, w1m[...]) + b1[...], 0.0)
    h = jnp.maximum(dot(h, w2[...]) + b2[...], 0.0)
    h = jnp.maximum(dot(h, w3[...]) + b3[...], 0.0)
    h = jnp.maximum(dot(h, w4[...]) + b4[...], 0.0)
    mf = sel(ue_mf[...], up) * sel(me_mf[...], mp)
    p = (jnp.sum(mf * wfa[...], axis=1) + jnp.sum(h * wfb[...], axis=1)
         + bfv[0, 0])
    out_ref[...] = 1.0 / (1.0 + jnp.exp(-p))


def _full(shape):
    return pl.BlockSpec(shape, lambda i: tuple(0 for _ in shape))


_tc_mlp = pl.pallas_call(
    _tc_body,
    grid=(BATCH // BB,),
    in_specs=[
        pl.BlockSpec((BB, DR), lambda i: (i, 0)),
        pl.BlockSpec((BB, DR), lambda i: (i, 0)),
        pl.BlockSpec((BB, DR), lambda i: (i, 0)),
        pl.BlockSpec((BB, DR), lambda i: (i, 0)),
        pl.BlockSpec((BB, 1), lambda i: (i, 0)),
        pl.BlockSpec((BB, 1), lambda i: (i, 0)),
        _full((D, 128)), _full((D, 128)), _full((1, 128)),
        _full((128, D)), _full((1, D)),
        _full((D, 32)), _full((1, 32)),
        _full((32, 16)), _full((1, 16)),
        _full((1, D)), _full((1, 16)), _full((1, 1)),
    ],
    out_specs=pl.BlockSpec((BB,), lambda i: (i,)),
    out_shape=jax.ShapeDtypeStruct((BATCH,), jnp.float32),
)


@jax.jit
def kernel(user_indices, movie_indices, Eu_mf, Em_mf, Eu_mlp, Em_mlp,
           W1, b1, W2, b2, W3, b3, W4, b4, Wf, bf):
    ui = user_indices.astype(jnp.int32)
    mi = movie_indices.astype(jnp.int32)
    uidx = (ui >> 1).reshape(NW, NCH, CHUNK)
    midx = (mi >> 1).reshape(NW, NCH, CHUNK)
    pu = (ui & 1).astype(jnp.float32).reshape(BATCH, 1)
    pm = (mi & 1).astype(jnp.float32).reshape(BATCH, 1)
    ue_mf, me_mf, ue_mlp, me_mlp = _sc_gather(
        uidx, midx,
        Eu_mf.reshape(-1, DR), Em_mf.reshape(-1, DR),
        Eu_mlp.reshape(-1, DR), Em_mlp.reshape(-1, DR))
    return _tc_mlp(
        ue_mf.reshape(BATCH, DR), me_mf.reshape(BATCH, DR),
        ue_mlp.reshape(BATCH, DR), me_mlp.reshape(BATCH, DR),
        pu, pm,
        W1[:D], W1[D:], b1.reshape(1, 128),
        W2, b2.reshape(1, D),
        W3, b3.reshape(1, 32),
        W4, b4.reshape(1, 16),
        Wf[:D, 0].reshape(1, D), Wf[D:, 0].reshape(1, 16),
        (bf.reshape(1, 1)).astype(jnp.float32),
    )


# speedup vs baseline: 1.5148x; 1.5071x over previous
import functools

import jax
import jax.numpy as jnp
from jax import lax
from jax.experimental import pallas as pl
from jax.experimental.pallas import tpu as pltpu
from jax.experimental.pallas import tpu_sc as plsc

BATCH = 16384
D = 64
NC = 2
NS = 16
NW = NC * NS
BPW = BATCH // NW
NTAB = 4

_sc_mesh = plsc.VectorSubcoreMesh(core_axis_name="c", subcore_axis_name="s")


HALF = BPW // 2


@functools.partial(
    pl.kernel,
    out_type=[jax.ShapeDtypeStruct((BATCH, D), jnp.float32)] * NTAB,
    mesh=_sc_mesh,
    compiler_params=pltpu.CompilerParams(use_tc_tiling_on_sc=True,
                                         needs_layout_passes=False),
    scratch_types=[
        pltpu.VMEM((BPW,), jnp.int32),
        pltpu.VMEM((BPW,), jnp.int32),
        pltpu.VMEM((HALF, D), jnp.float32),
        pltpu.VMEM((HALF, D), jnp.float32),
        pltpu.SemaphoreType.DMA,
    ],
)
def _sc_gather(uidx_hbm, midx_hbm, eu_mf, em_mf, eu_mlp, em_mlp,
               ue_mf_out, me_mf_out, ue_mlp_out, me_mlp_out,
               vu, vm, buf_a, buf_b, gsem):
    wid = lax.axis_index("s") * NC + lax.axis_index("c")
    r0 = wid * BPW

    pltpu.sync_copy(uidx_hbm.at[wid], vu)
    pltpu.sync_copy(midx_hbm.at[wid], vm)

    iota16 = lax.iota(jnp.int32, 16)

    for vidx, tab0, tab1, out0, out1 in (
        (vu, eu_mf, eu_mlp, ue_mf_out, ue_mlp_out),
        (vm, em_mf, em_mlp, me_mf_out, me_mlp_out),
    ):
        for h in range(2):
            base = h * HALF

            def enq(g, carry):
                vals = vidx[pl.ds(base + g * 16, 16)]
                for j in range(16):
                    s = jnp.max(jnp.where(iota16 == j, vals, 0))
                    l = g * 16 + j
                    pltpu.async_copy(tab0.at[s], buf_a.at[l], gsem)
                    pltpu.async_copy(tab1.at[s], buf_b.at[l], gsem)
                return carry
            lax.fori_loop(0, HALF // 16, enq, 0)

            def drain(l, carry):
                pltpu.make_async_copy(tab0.at[0], buf_a.at[l], gsem).wait()
                pltpu.make_async_copy(tab1.at[0], buf_b.at[l], gsem).wait()
                return carry
            lax.fori_loop(0, HALF, drain, 0)

            pltpu.sync_copy(buf_a, out0.at[pl.ds(r0 + base, HALF)])
            pltpu.sync_copy(buf_b, out1.at[pl.ds(r0 + base, HALF)])


BB = 2048


def _tc_body(ue_mf, me_mf, ue_mlp, me_mlp, w1u, w1m, b1, w2, b2,
             w3, b3, w4, b4, wfa, wfb, bfv, out_ref):
    dot = functools.partial(jnp.dot, preferred_element_type=jnp.float32)
    h = jnp.maximum(dot(ue_mlp[...], w1u[...]) + dot(me_mlp[...], w1m[...])
                    + b1[...], 0.0)
    h = jnp.maximum(dot(h, w2[...]) + b2[...], 0.0)
    h = jnp.maximum(dot(h, w3[...]) + b3[...], 0.0)
    h = jnp.maximum(dot(h, w4[...]) + b4[...], 0.0)
    mf = ue_mf[...] * me_mf[...]
    p = (jnp.sum(mf * wfa[...], axis=1) + jnp.sum(h * wfb[...], axis=1)
         + bfv[0, 0])
    out_ref[...] = 1.0 / (1.0 + jnp.exp(-p))


def _full(shape):
    return pl.BlockSpec(shape, lambda i: tuple(0 for _ in shape))


_tc_mlp = pl.pallas_call(
    _tc_body,
    grid=(BATCH // BB,),
    in_specs=[
        pl.BlockSpec((BB, D), lambda i: (i, 0)),
        pl.BlockSpec((BB, D), lambda i: (i, 0)),
        pl.BlockSpec((BB, D), lambda i: (i, 0)),
        pl.BlockSpec((BB, D), lambda i: (i, 0)),
        _full((D, 128)), _full((D, 128)), _full((1, 128)),
        _full((128, D)), _full((1, D)),
        _full((D, 32)), _full((1, 32)),
        _full((32, 16)), _full((1, 16)),
        _full((1, D)), _full((1, 16)), _full((1, 1)),
    ],
    out_specs=pl.BlockSpec((BB,), lambda i: (i,)),
    out_shape=jax.ShapeDtypeStruct((BATCH,), jnp.float32),
)


@jax.jit
def kernel(user_indices, movie_indices, Eu_mf, Em_mf, Eu_mlp, Em_mlp,
           W1, b1, W2, b2, W3, b3, W4, b4, Wf, bf):
    uidx = user_indices.astype(jnp.int32).reshape(NW, BPW)
    midx = movie_indices.astype(jnp.int32).reshape(NW, BPW)
    ue_mf, me_mf, ue_mlp, me_mlp = _sc_gather(
        uidx, midx, Eu_mf, Em_mf, Eu_mlp, Em_mlp)
    return _tc_mlp(
        ue_mf, me_mf, ue_mlp, me_mlp,
        W1[:D], W1[D:], b1.reshape(1, 128),
        W2, b2.reshape(1, D),
        W3, b3.reshape(1, 32),
        W4, b4.reshape(1, 16),
        Wf[:D, 0].reshape(1, D), Wf[D:, 0].reshape(1, 16),
        (bf.reshape(1, 1)).astype(jnp.float32),
    )


# speedup vs baseline: 1.5162x; 1.0009x over previous
import functools

import jax
import jax.numpy as jnp
from jax import lax
from jax.experimental import pallas as pl
from jax.experimental.pallas import tpu as pltpu
from jax.experimental.pallas import tpu_sc as plsc

BATCH = 16384
D = 64
NC = 2
NS = 16
NW = NC * NS
BPW = BATCH // NW
NTAB = 4

_sc_mesh = plsc.VectorSubcoreMesh(core_axis_name="c", subcore_axis_name="s")


HALF = BPW // 2


@functools.partial(
    pl.kernel,
    out_type=[jax.ShapeDtypeStruct((BATCH, D), jnp.float32)] * NTAB,
    mesh=_sc_mesh,
    compiler_params=pltpu.CompilerParams(use_tc_tiling_on_sc=True),
    scratch_types=[
        pltpu.VMEM((BPW,), jnp.int32),
        pltpu.VMEM((BPW,), jnp.int32),
        pltpu.VMEM((HALF, D), jnp.float32),
        pltpu.VMEM((HALF, D), jnp.float32),
        pltpu.SemaphoreType.DMA,
    ],
)
def _sc_gather(uidx_hbm, midx_hbm, eu_mf, em_mf, eu_mlp, em_mlp,
               ue_mf_out, me_mf_out, ue_mlp_out, me_mlp_out,
               vu, vm, buf_a, buf_b, gsem):
    wid = lax.axis_index("s") * NC + lax.axis_index("c")
    r0 = wid * BPW

    pltpu.sync_copy(uidx_hbm.at[wid], vu)
    pltpu.sync_copy(midx_hbm.at[wid], vm)

    iota16 = lax.iota(jnp.int32, 16)

    for vidx, tab0, tab1, out0, out1 in (
        (vu, eu_mf, eu_mlp, ue_mf_out, ue_mlp_out),
        (vm, em_mf, em_mlp, me_mf_out, me_mlp_out),
    ):
        for h in range(2):
            base = h * HALF

            def enq(g, carry):
                vals = vidx[pl.ds(base + g * 16, 16)]
                for j in range(16):
                    s = vals[j]
                    l = g * 16 + j
                    pltpu.async_copy(tab0.at[s], buf_a.at[l], gsem)
                    pltpu.async_copy(tab1.at[s], buf_b.at[l], gsem)
                return carry
            lax.fori_loop(0, HALF // 16, enq, 0)

            def drain(l, carry):
                pltpu.make_async_copy(tab0.at[0], buf_a.at[l], gsem).wait()
                pltpu.make_async_copy(tab1.at[0], buf_b.at[l], gsem).wait()
                return carry
            lax.fori_loop(0, HALF, drain, 0)

            pltpu.sync_copy(buf_a, out0.at[pl.ds(r0 + base, HALF)])
            pltpu.sync_copy(buf_b, out1.at[pl.ds(r0 + base, HALF)])


BB = 2048


def _tc_body(ue_mf, me_mf, ue_mlp, me_mlp, w1u, w1m, b1, w2, b2,
             w3, b3, w4, b4, wfa, wfb, bfv, out_ref):
    dot = functools.partial(jnp.dot, preferred_element_type=jnp.float32)
    h = jnp.maximum(dot(ue_mlp[...], w1u[...]) + dot(me_mlp[...], w1m[...])
                    + b1[...], 0.0)
    h = jnp.maximum(dot(h, w2[...]) + b2[...], 0.0)
    h = jnp.maximum(dot(h, w3[...]) + b3[...], 0.0)
    h = jnp.maximum(dot(h, w4[...]) + b4[...], 0.0)
    mf = ue_mf[...] * me_mf[...]
    p = (jnp.sum(mf * wfa[...], axis=1) + jnp.sum(h * wfb[...], axis=1)
         + bfv[0, 0])
    out_ref[...] = 1.0 / (1.0 + jnp.exp(-p))


def _full(shape):
    return pl.BlockSpec(shape, lambda i: tuple(0 for _ in shape))


_tc_mlp = pl.pallas_call(
    _tc_body,
    grid=(BATCH // BB,),
    in_specs=[
        pl.BlockSpec((BB, D), lambda i: (i, 0)),
        pl.BlockSpec((BB, D), lambda i: (i, 0)),
        pl.BlockSpec((BB, D), lambda i: (i, 0)),
        pl.BlockSpec((BB, D), lambda i: (i, 0)),
        _full((D, 128)), _full((D, 128)), _full((1, 128)),
        _full((128, D)), _full((1, D)),
        _full((D, 32)), _full((1, 32)),
        _full((32, 16)), _full((1, 16)),
        _full((1, D)), _full((1, 16)), _full((1, 1)),
    ],
    out_specs=pl.BlockSpec((BB,), lambda i: (i,)),
    out_shape=jax.ShapeDtypeStruct((BATCH,), jnp.float32),
)


@jax.jit
def kernel(user_indices, movie_indices, Eu_mf, Em_mf, Eu_mlp, Em_mlp,
           W1, b1, W2, b2, W3, b3, W4, b4, Wf, bf):
    uidx = user_indices.astype(jnp.int32).reshape(NW, BPW)
    midx = movie_indices.astype(jnp.int32).reshape(NW, BPW)
    ue_mf, me_mf, ue_mlp, me_mlp = _sc_gather(
        uidx, midx, Eu_mf, Em_mf, Eu_mlp, Em_mlp)
    return _tc_mlp(
        ue_mf, me_mf, ue_mlp, me_mlp,
        W1[:D], W1[D:], b1.reshape(1, 128),
        W2, b2.reshape(1, D),
        W3, b3.reshape(1, 32),
        W4, b4.reshape(1, 16),
        Wf[:D, 0].reshape(1, D), Wf[D:, 0].reshape(1, 16),
        (bf.reshape(1, 1)).astype(jnp.float32),
    )
